# baseline (speedup 1.0000x reference)
import functools

import jax
import jax.numpy as jnp
from jax import lax
from jax.experimental import pallas as pl
from jax.experimental.pallas import tpu as pltpu
from jax.experimental.pallas import tpu_sc as plsc

N = 10000
D = 128
H = 256
HH = 128
G = 64
L = 4
BN_EPS = 1e-5

NSC = 2
NSUB = 16
CHUNK = 128

R = 2560
EPAD = R * CHUNK
RW_DEG = R // (NSC * NSUB)
RS_PROP = R // NSUB
IGRP = 8
PGRP = 16
NBUF = 2

NPAD = 10240
ZROWS = NPAD // NSUB

BN_TC = 1000
BB = 1000

_mesh = plsc.VectorSubcoreMesh(core_axis_name="c", subcore_axis_name="s")


@jax.jit
def _sc_degree(dst2d):
    @functools.partial(
        pl.kernel,
        mesh=_mesh,
        out_type=jax.ShapeDtypeStruct((NSC, NPAD, HH), jnp.float32),
        scratch_types=[
            pltpu.VMEM((IGRP, CHUNK), jnp.int32),
            pltpu.VMEM((CHUNK, HH), jnp.float32),
            pltpu.VMEM((16, HH), jnp.float32),
            pltpu.VMEM_SHARED((NPAD, HH), jnp.float32),
        ],
    )
    def deg_k(dst_hbm, o_hbm, idxb, onesb, zb, acc):
        cid = lax.axis_index("c")
        sid = lax.axis_index("s")
        ones16 = jnp.ones((16,), jnp.float32)
        zero16 = jnp.zeros((16,), jnp.float32)

        @pl.loop(0, CHUNK)
        def _(i):
            @pl.loop(0, HH, step=16)
            def _(j):
                onesb[i, pl.ds(j, 16)] = ones16

        @pl.loop(0, 16)
        def _(i):
            @pl.loop(0, HH, step=16)
            def _(j):
                zb[i, pl.ds(j, 16)] = zero16

        zbase = sid * ZROWS

        @pl.loop(0, ZROWS, step=16)
        def _(i):
            pltpu.sync_copy(zb, acc.at[pl.ds(zbase + i, 16)])

        plsc.subcore_barrier()

        wid = cid * NSUB + sid
        rowbase = wid * RW_DEG

        @pl.loop(0, RW_DEG, step=IGRP)
        def _(g):
            pltpu.sync_copy(dst_hbm.at[pl.ds(rowbase + g, IGRP)], idxb)
            for j in range(IGRP):
                pltpu.sync_copy(onesb, acc.at[idxb.at[j]], add=True)

        plsc.subcore_barrier()
        wbase = sid * ZROWS
        pltpu.sync_copy(acc.at[pl.ds(wbase, ZROWS)],
                        o_hbm.at[cid, pl.ds(wbase, ZROWS)])

    return deg_k(dst2d)


@jax.jit
def _sc_prop(y0, y1, src2d, dst2d):
    @functools.partial(
        pl.kernel,
        mesh=_mesh,
        out_type=(
            jax.ShapeDtypeStruct((NPAD, HH), jnp.float32),
            jax.ShapeDtypeStruct((NPAD, HH), jnp.float32),
        ),
        scratch_types=[
            pltpu.VMEM((PGRP, CHUNK), jnp.int32),
            pltpu.VMEM((PGRP, CHUNK), jnp.int32),
            pltpu.VMEM((CHUNK, HH), jnp.float32),
            pltpu.VMEM((CHUNK, HH), jnp.float32),
            pltpu.VMEM((16, HH), jnp.float32),
            pltpu.VMEM_SHARED((NPAD, HH), jnp.float32),
            pltpu.SemaphoreType.DMA,
            pltpu.SemaphoreType.DMA,
            pltpu.SemaphoreType.DMA,
            pltpu.SemaphoreType.DMA,
            pltpu.SemaphoreType.DMA,
            pltpu.SemaphoreType.DMA,
            pltpu.SemaphoreType.DMA,
        ],
    )
    def prop_k(y0_hbm, y1_hbm, src_hbm, dst_hbm, s0_hbm, s1_hbm,
               sib, dib, r0, r1, zb, acc,
               g0, g1, g2, g3, s0, s1, zsem):
        cid = lax.axis_index("c")
        sid = lax.axis_index("s")
        rows = [r0, r1]
        gsem = [[g0, g1], [g2, g3]]
        ssem = [s0, s1]
        zero16 = jnp.zeros((16,), jnp.float32)

        @pl.loop(0, 16)
        def _(i):
            @pl.loop(0, HH, step=16)
            def _(j):
                zb[i, pl.ds(j, 16)] = zero16

        zbase = sid * ZROWS
        zcp = [pltpu.async_copy(zb, acc.at[pl.ds(zbase + 16 * i, 16)], zsem)
               for i in range(ZROWS // 16)]
        for c in zcp:
            c.wait()

        plsc.subcore_barrier()
        rowbase = sid * RS_PROP

        def run(y_hbm, o_hbm):
            @pl.loop(0, RS_PROP, step=PGRP)
            def _(g):
                pltpu.sync_copy(src_hbm.at[pl.ds(rowbase + g, PGRP)], sib)
                pltpu.sync_copy(dst_hbm.at[pl.ds(rowbase + g, PGRP)], dib)
                gcp = [None] * PGRP
                scp = [None] * PGRP
                HC = CHUNK // 2

                def start_gather(j):
                    b = j % NBUF
                    ga = pltpu.async_copy(
                        y_hbm.at[sib.at[j, pl.ds(0, HC)]],
                        rows[b].at[pl.ds(0, HC)], gsem[b][0])
                    gb = pltpu.async_copy(
                        y_hbm.at[sib.at[j, pl.ds(HC, HC)]],
                        rows[b].at[pl.ds(HC, HC)], gsem[b][1])
                    return (ga, gb)

                for j in range(PGRP):
                    b = j % NBUF
                    if j >= NBUF:
                        scp[j - NBUF].wait()
                    gcp[j] = start_gather(j)
                    if j >= 1:
                        pb = (j - 1) % NBUF
                        gcp[j - 1][0].wait()
                        gcp[j - 1][1].wait()
                        scp[j - 1] = pltpu.async_copy(
                            rows[pb], acc.at[dib.at[j - 1]], ssem[pb],
                            add=True)
                lb = (PGRP - 1) % NBUF
                gcp[PGRP - 1][0].wait()
                gcp[PGRP - 1][1].wait()
                scp[PGRP - 1] = pltpu.async_copy(
                    rows[lb], acc.at[dib.at[PGRP - 1]], ssem[lb], add=True)
                for j in range(PGRP - NBUF, PGRP):
                    scp[j].wait()

            plsc.subcore_barrier()
            wbase = sid * ZROWS
            pltpu.sync_copy(acc.at[pl.ds(wbase, ZROWS)],
                            o_hbm.at[pl.ds(wbase, ZROWS)])

        @pl.when(cid == 0)
        def _():
            run(y0_hbm, s0_hbm)

        @pl.when(cid == 1)
        def _():
            run(y1_hbm, s1_hbm)

    return prop_k(y0, y1, src2d, dst2d)


def _tc_encode(x, degp0, degp1, enc_W, enc_b, W0):
    nb = N // BN_TC

    def body(x_r, d0_r, d1_r, ew_r, eb_r, w0_r, h_r, d16_r, y0_r, y1_r):
        xb = x_r[...]
        h = jnp.dot(xb, ew_r[...], preferred_element_type=jnp.float32,
                     precision=lax.Precision.HIGHEST)
        h = h + eb_r[...]
        deg = d0_r[...][:, :16] + d1_r[...][:, :16] + 1.0
        dv16 = lax.rsqrt(deg)
        dv = dv16[:, :1]
        h_r[...] = h
        d16_r[...] = dv16
        xw = jnp.dot(h, w0_r[...], preferred_element_type=jnp.float32,
                     precision=lax.Precision.HIGHEST)
        y = dv * xw
        y0_r[...] = y[:, :HH]
        y1_r[...] = y[:, HH:]

    return pl.pallas_call(
        body,
        grid=(nb,),
        in_specs=[
            pl.BlockSpec((BN_TC, D), lambda i: (i, 0)),
            pl.BlockSpec((BN_TC, HH), lambda i: (i, 0)),
            pl.BlockSpec((BN_TC, HH), lambda i: (i, 0)),
            pl.BlockSpec((D, H), lambda i: (0, 0)),
            pl.BlockSpec((1, H), lambda i: (0, 0)),
            pl.BlockSpec((H, H), lambda i: (0, 0)),
        ],
        out_specs=[
            pl.BlockSpec((BN_TC, H), lambda i: (i, 0)),
            pl.BlockSpec((BN_TC, 16), lambda i: (i, 0)),
            pl.BlockSpec((BN_TC, HH), lambda i: (i, 0)),
            pl.BlockSpec((BN_TC, HH), lambda i: (i, 0)),
        ],
        out_shape=[
            jax.ShapeDtypeStruct((N, H), jnp.float32),
            jax.ShapeDtypeStruct((N, 16), jnp.float32),
            jax.ShapeDtypeStruct((N, HH), jnp.float32),
            jax.ShapeDtypeStruct((N, HH), jnp.float32),
        ],
    )(x, degp0, degp1, enc_W, enc_b.reshape(1, H), W0)


def _tc_combine(h, y0, y1, s0, s1, d16, b, sg, bb, Wn):
    nb = N // BN_TC
    has_next = Wn is not None

    def body(h_r, y0_r, y1_r, s0_r, s1_r, d16_r, b_r, sg_r, bb_r, *rest):
        if has_next:
            wn_r, hn_r, yn0_r, yn1_r = rest
        else:
            (hn_r,) = rest
        dv = d16_r[...][:, :1]
        agg = jnp.concatenate([s0_r[...] + y0_r[...],
                               s1_r[...] + y1_r[...]], axis=1)
        t = dv * agg + b_r[...]
        t = jnp.maximum(t, 0.0) * sg_r[...] + bb_r[...]
        hn = h_r[...] + t
        hn_r[...] = hn
        if has_next:
            xw = jnp.dot(hn, wn_r[...], preferred_element_type=jnp.float32,
                     precision=lax.Precision.HIGHEST)
            y = dv * xw
            yn0_r[...] = y[:, :HH]
            yn1_r[...] = y[:, HH:]

    in_specs = [
        pl.BlockSpec((BN_TC, H), lambda i: (i, 0)),
        pl.BlockSpec((BN_TC, HH), lambda i: (i, 0)),
        pl.BlockSpec((BN_TC, HH), lambda i: (i, 0)),
        pl.BlockSpec((BN_TC, HH), lambda i: (i, 0)),
        pl.BlockSpec((BN_TC, HH), lambda i: (i, 0)),
        pl.BlockSpec((BN_TC, 16), lambda i: (i, 0)),
        pl.BlockSpec((1, H), lambda i: (0, 0)),
        pl.BlockSpec((1, H), lambda i: (0, 0)),
        pl.BlockSpec((1, H), lambda i: (0, 0)),
    ]
    out_specs = [pl.BlockSpec((BN_TC, H), lambda i: (i, 0))]
    out_shape = [jax.ShapeDtypeStruct((N, H), jnp.float32)]
    args = [h, y0, y1, s0, s1, d16, b.reshape(1, H), sg.reshape(1, H),
            bb.reshape(1, H)]
    if has_next:
        in_specs.append(pl.BlockSpec((H, H), lambda i: (0, 0)))
        out_specs += [
            pl.BlockSpec((BN_TC, HH), lambda i: (i, 0)),
            pl.BlockSpec((BN_TC, HH), lambda i: (i, 0)),
        ]
        out_shape += [
            jax.ShapeDtypeStruct((N, HH), jnp.float32),
            jax.ShapeDtypeStruct((N, HH), jnp.float32),
        ]
        args.append(Wn)

    return pl.pallas_call(
        body,
        grid=(nb,),
        in_specs=in_specs,
        out_specs=out_specs,
        out_shape=out_shape,
    )(*args)


def _tc_pool_head(h, batch3, W1, b1, W2row, b2):
    nb = N // BB

    def body(h_r, bt_r, w1_r, b1_r, w2_r, b2_r, o_r, s_acc, c_acc):
        step = pl.program_id(0)

        @pl.when(step == 0)
        def _():
            s_acc[...] = jnp.zeros_like(s_acc)
            c_acc[...] = jnp.zeros_like(c_acc)

        bt = bt_r[...].reshape(1, BB)
        gid = lax.broadcasted_iota(jnp.int32, (G, BB), 0)
        oh = (gid == bt).astype(jnp.float32)
        s_acc[...] += jnp.dot(oh, h_r[...], preferred_element_type=jnp.float32,
                     precision=lax.Precision.HIGHEST)
        c_acc[...] += jnp.broadcast_to(
            jnp.sum(oh, axis=1, keepdims=True), (G, 128))

        @pl.when(step == nb - 1)
        def _():
            s = s_acc[...]
            cnt = jnp.maximum(c_acc[...][:, :1], 1.0)
            gvec = jnp.concatenate([s / cnt, s], axis=1)
            t = jnp.dot(gvec, w1_r[...], preferred_element_type=jnp.float32,
                     precision=lax.Precision.HIGHEST)
            t = jnp.maximum(t + b1_r[...], 0.0)
            o = jnp.sum(t * w2_r[...], axis=1, keepdims=True) + b2_r[...]
            o_r[...] = o

    return pl.pallas_call(
        body,
        grid=(nb,),
        in_specs=[
            pl.BlockSpec((BB, H), lambda i: (i, 0)),
            pl.BlockSpec((1, 1, BB), lambda i: (i, 0, 0)),
            pl.BlockSpec((2 * H, H), lambda i: (0, 0)),
            pl.BlockSpec((1, H), lambda i: (0, 0)),
            pl.BlockSpec((1, H), lambda i: (0, 0)),
            pl.BlockSpec((1, 1), lambda i: (0, 0)),
        ],
        out_specs=pl.BlockSpec((G, 1), lambda i: (0, 0)),
        out_shape=jax.ShapeDtypeStruct((G, 1), jnp.float32),
        scratch_shapes=[
            pltpu.VMEM((G, H), jnp.float32),
            pltpu.VMEM((G, 128), jnp.float32),
        ],
    )(h, batch3, W1, b1.reshape(1, H), W2row, b2.reshape(1, 1))


def kernel(x, edge_index, batch, enc_W, enc_b, conv_W, conv_b, bn_g, bn_b,
           head_W1, head_b1, head_W2, head_b2):
    src = edge_index[0]
    dst = edge_index[1]
    pad = EPAD - src.shape[0]
    src_p = jnp.concatenate([src, jnp.zeros((pad,), jnp.int32)])
    dst_p = jnp.concatenate([dst, jnp.full((pad,), N, jnp.int32)])
    src2d = src_p.reshape(R, CHUNK)
    dst2d = dst_p.reshape(R, CHUNK)
    batch3 = batch.reshape(N // BB, 1, BB)
    sg = bn_g / jnp.sqrt(1.0 + BN_EPS)

    degp = _sc_degree(dst2d)
    degp0, degp1 = degp[0], degp[1]
    h, d16, y0, y1 = _tc_encode(x, degp0, degp1, enc_W, enc_b, conv_W[0])
    for i in range(L):
        s0, s1 = _sc_prop(y0, y1, src2d, dst2d)
        Wn = conv_W[i + 1] if i + 1 < L else None
        outs = _tc_combine(h, y0, y1, s0, s1, d16, conv_b[i], sg[i], bn_b[i],
                           Wn)
        if Wn is not None:
            h, y0, y1 = outs
        else:
            (h,) = outs

    o = _tc_pool_head(h, batch3, head_W1, head_b1,
                      head_W2.reshape(1, H), head_b2)
    return o[:, 0]


# speedup vs baseline: 1.0137x; 1.0137x over previous
import functools

import jax
import jax.numpy as jnp
from jax import lax
from jax.experimental import pallas as pl
from jax.experimental.pallas import tpu as pltpu
from jax.experimental.pallas import tpu_sc as plsc

N = 10000
D = 128
H = 256
HH = 128
G = 64
L = 4
BN_EPS = 1e-5

NSC = 2
NSUB = 16
CHUNK = 128

R = 2560
EPAD = R * CHUNK
RW_DEG = R // (NSC * NSUB)
RS_PROP = R // NSUB
IGRP = 8
PGRP = 16
NBUF = 2

NPAD = 10240
ZROWS = NPAD // NSUB

BN_TC = 1000
BB = 1000

_mesh = plsc.VectorSubcoreMesh(core_axis_name="c", subcore_axis_name="s")


@jax.jit
def _sc_degree(dst2d):
    @functools.partial(
        pl.kernel,
        mesh=_mesh,
        out_type=jax.ShapeDtypeStruct((NSC, NPAD, HH), jnp.float32),
        scratch_types=[
            pltpu.VMEM((IGRP, CHUNK), jnp.int32),
            pltpu.VMEM((CHUNK, HH), jnp.float32),
            pltpu.VMEM((16, HH), jnp.float32),
            pltpu.VMEM_SHARED((NPAD, HH), jnp.float32),
        ],
    )
    def deg_k(dst_hbm, o_hbm, idxb, onesb, zb, acc):
        cid = lax.axis_index("c")
        sid = lax.axis_index("s")
        ones16 = jnp.ones((16,), jnp.float32)
        zero16 = jnp.zeros((16,), jnp.float32)

        @pl.loop(0, CHUNK)
        def _(i):
            @pl.loop(0, HH, step=16)
            def _(j):
                onesb[i, pl.ds(j, 16)] = ones16

        @pl.loop(0, 16)
        def _(i):
            @pl.loop(0, HH, step=16)
            def _(j):
                zb[i, pl.ds(j, 16)] = zero16

        zbase = sid * ZROWS

        @pl.loop(0, ZROWS, step=16)
        def _(i):
            pltpu.sync_copy(zb, acc.at[pl.ds(zbase + i, 16)])

        plsc.subcore_barrier()

        wid = cid * NSUB + sid
        rowbase = wid * RW_DEG

        @pl.loop(0, RW_DEG, step=IGRP)
        def _(g):
            pltpu.sync_copy(dst_hbm.at[pl.ds(rowbase + g, IGRP)], idxb)
            for j in range(IGRP):
                pltpu.sync_copy(onesb, acc.at[idxb.at[j]], add=True)

        plsc.subcore_barrier()
        wbase = sid * ZROWS
        pltpu.sync_copy(acc.at[pl.ds(wbase, ZROWS)],
                        o_hbm.at[cid, pl.ds(wbase, ZROWS)])

    return deg_k(dst2d)


@jax.jit
def _sc_prop(y0, y1, src2d, dst2d):
    @functools.partial(
        pl.kernel,
        mesh=_mesh,
        out_type=(
            jax.ShapeDtypeStruct((NPAD, HH), jnp.float32),
            jax.ShapeDtypeStruct((NPAD, HH), jnp.float32),
        ),
        scratch_types=[
            pltpu.VMEM((PGRP, CHUNK), jnp.int32),
            pltpu.VMEM((PGRP, CHUNK), jnp.int32),
            pltpu.VMEM((PGRP, CHUNK), jnp.int32),
            pltpu.VMEM((PGRP, CHUNK), jnp.int32),
            pltpu.VMEM((CHUNK, HH), jnp.float32),
            pltpu.VMEM((CHUNK, HH), jnp.float32),
            pltpu.VMEM((16, HH), jnp.float32),
            pltpu.VMEM_SHARED((NPAD, HH), jnp.float32),
            pltpu.SemaphoreType.DMA,
            pltpu.SemaphoreType.DMA,
            pltpu.SemaphoreType.DMA,
            pltpu.SemaphoreType.DMA,
            pltpu.SemaphoreType.DMA,
            pltpu.SemaphoreType.DMA,
        ],
    )
    def prop_k(y0_hbm, y1_hbm, src_hbm, dst_hbm, s0_hbm, s1_hbm,
               sib0, dib0, sib1, dib1, r0, r1, zb, acc,
               g0, g1, s0, s1, isem, zsem):
        cid = lax.axis_index("c")
        sid = lax.axis_index("s")
        rows = [r0, r1]
        gsem = [g0, g1]
        ssem = [s0, s1]
        sibs = [sib0, sib1]
        dibs = [dib0, dib1]
        zero16 = jnp.zeros((16,), jnp.float32)

        @pl.loop(0, 16)
        def _(i):
            @pl.loop(0, HH, step=16)
            def _(j):
                zb[i, pl.ds(j, 16)] = zero16

        zbase = sid * ZROWS
        zcp = [pltpu.async_copy(zb, acc.at[pl.ds(zbase + 16 * i, 16)], zsem)
               for i in range(ZROWS // 16)]
        for c in zcp:
            c.wait()

        plsc.subcore_barrier()
        rowbase = sid * RS_PROP

        def run(y_hbm, o_hbm):
            def process(sib, dib):
                gcp = [None] * PGRP
                scp = [None] * PGRP
                for j in range(PGRP):
                    b = j % NBUF
                    if j >= NBUF:
                        scp[j - NBUF].wait()
                    gcp[j] = pltpu.async_copy(y_hbm.at[sib.at[j]], rows[b],
                                              gsem[b])
                    if j >= 1:
                        pb = (j - 1) % NBUF
                        gcp[j - 1].wait()
                        scp[j - 1] = pltpu.async_copy(
                            rows[pb], acc.at[dib.at[j - 1]], ssem[pb],
                            add=True)
                lb = (PGRP - 1) % NBUF
                gcp[PGRP - 1].wait()
                scp[PGRP - 1] = pltpu.async_copy(
                    rows[lb], acc.at[dib.at[PGRP - 1]], ssem[lb], add=True)
                for j in range(PGRP - NBUF, PGRP):
                    scp[j].wait()

            def prefetch(row, bi):
                ca = pltpu.async_copy(src_hbm.at[pl.ds(row, PGRP)],
                                      sibs[bi], isem)
                cb = pltpu.async_copy(dst_hbm.at[pl.ds(row, PGRP)],
                                      dibs[bi], isem)
                return (ca, cb)

            pltpu.sync_copy(src_hbm.at[pl.ds(rowbase, PGRP)], sibs[0])
            pltpu.sync_copy(dst_hbm.at[pl.ds(rowbase, PGRP)], dibs[0])
            p = prefetch(rowbase + PGRP, 1)

            @pl.loop(0, RS_PROP, step=2 * PGRP)
            def _(g):
                process(sibs[0], dibs[0])
                nxt = pl.multiple_of(
                    jnp.minimum(rowbase + g + 2 * PGRP,
                                rowbase + RS_PROP - PGRP), PGRP)
                pltpu.make_async_copy(src_hbm.at[pl.ds(nxt, PGRP)],
                                      sibs[0], isem).wait()
                pltpu.make_async_copy(dst_hbm.at[pl.ds(nxt, PGRP)],
                                      dibs[0], isem).wait()
                prefetch(nxt, 0)
                process(sibs[1], dibs[1])
                nxt2 = pl.multiple_of(
                    jnp.minimum(rowbase + g + 3 * PGRP,
                                rowbase + RS_PROP - PGRP), PGRP)
                pltpu.make_async_copy(src_hbm.at[pl.ds(nxt2, PGRP)],
                                      sibs[1], isem).wait()
                pltpu.make_async_copy(dst_hbm.at[pl.ds(nxt2, PGRP)],
                                      dibs[1], isem).wait()
                prefetch(nxt2, 1)

            pltpu.make_async_copy(src_hbm.at[pl.ds(rowbase, PGRP)],
                                  sibs[1], isem).wait()
            pltpu.make_async_copy(dst_hbm.at[pl.ds(rowbase, PGRP)],
                                  dibs[1], isem).wait()
            plsc.subcore_barrier()
            wbase = sid * ZROWS
            pltpu.sync_copy(acc.at[pl.ds(wbase, ZROWS)],
                            o_hbm.at[pl.ds(wbase, ZROWS)])

        @pl.when(cid == 0)
        def _():
            run(y0_hbm, s0_hbm)

        @pl.when(cid == 1)
        def _():
            run(y1_hbm, s1_hbm)

    return prop_k(y0, y1, src2d, dst2d)


def _tc_encode(x, degp0, degp1, enc_W, enc_b, W0):
    nb = N // BN_TC

    def body(x_r, d0_r, d1_r, ew_r, eb_r, w0_r, h_r, d16_r, y0_r, y1_r):
        xb = x_r[...]
        h = jnp.dot(xb, ew_r[...], preferred_element_type=jnp.float32,
                     precision=lax.Precision.HIGHEST)
        h = h + eb_r[...]
        deg = d0_r[...][:, :16] + d1_r[...][:, :16] + 1.0
        dv16 = lax.rsqrt(deg)
        dv = dv16[:, :1]
        h_r[...] = h
        d16_r[...] = dv16
        xw = jnp.dot(h, w0_r[...], preferred_element_type=jnp.float32,
                     precision=lax.Precision.HIGHEST)
        y = dv * xw
        y0_r[...] = y[:, :HH]
        y1_r[...] = y[:, HH:]

    return pl.pallas_call(
        body,
        grid=(nb,),
        in_specs=[
            pl.BlockSpec((BN_TC, D), lambda i: (i, 0)),
            pl.BlockSpec((BN_TC, HH), lambda i: (i, 0)),
            pl.BlockSpec((BN_TC, HH), lambda i: (i, 0)),
            pl.BlockSpec((D, H), lambda i: (0, 0)),
            pl.BlockSpec((1, H), lambda i: (0, 0)),
            pl.BlockSpec((H, H), lambda i: (0, 0)),
        ],
        out_specs=[
            pl.BlockSpec((BN_TC, H), lambda i: (i, 0)),
            pl.BlockSpec((BN_TC, 16), lambda i: (i, 0)),
            pl.BlockSpec((BN_TC, HH), lambda i: (i, 0)),
            pl.BlockSpec((BN_TC, HH), lambda i: (i, 0)),
        ],
        out_shape=[
            jax.ShapeDtypeStruct((N, H), jnp.float32),
            jax.ShapeDtypeStruct((N, 16), jnp.float32),
            jax.ShapeDtypeStruct((N, HH), jnp.float32),
            jax.ShapeDtypeStruct((N, HH), jnp.float32),
        ],
    )(x, degp0, degp1, enc_W, enc_b.reshape(1, H), W0)


def _tc_combine(h, y0, y1, s0, s1, d16, b, sg, bb, Wn):
    nb = N // BN_TC
    has_next = Wn is not None

    def body(h_r, y0_r, y1_r, s0_r, s1_r, d16_r, b_r, sg_r, bb_r, *rest):
        if has_next:
            wn_r, hn_r, yn0_r, yn1_r = rest
        else:
            (hn_r,) = rest
        dv = d16_r[...][:, :1]
        agg = jnp.concatenate([s0_r[...] + y0_r[...],
                               s1_r[...] + y1_r[...]], axis=1)
        t = dv * agg + b_r[...]
        t = jnp.maximum(t, 0.0) * sg_r[...] + bb_r[...]
        hn = h_r[...] + t
        hn_r[...] = hn
        if has_next:
            xw = jnp.dot(hn, wn_r[...], preferred_element_type=jnp.float32,
                     precision=lax.Precision.HIGHEST)
            y = dv * xw
            yn0_r[...] = y[:, :HH]
            yn1_r[...] = y[:, HH:]

    in_specs = [
        pl.BlockSpec((BN_TC, H), lambda i: (i, 0)),
        pl.BlockSpec((BN_TC, HH), lambda i: (i, 0)),
        pl.BlockSpec((BN_TC, HH), lambda i: (i, 0)),
        pl.BlockSpec((BN_TC, HH), lambda i: (i, 0)),
        pl.BlockSpec((BN_TC, HH), lambda i: (i, 0)),
        pl.BlockSpec((BN_TC, 16), lambda i: (i, 0)),
        pl.BlockSpec((1, H), lambda i: (0, 0)),
        pl.BlockSpec((1, H), lambda i: (0, 0)),
        pl.BlockSpec((1, H), lambda i: (0, 0)),
    ]
    out_specs = [pl.BlockSpec((BN_TC, H), lambda i: (i, 0))]
    out_shape = [jax.ShapeDtypeStruct((N, H), jnp.float32)]
    args = [h, y0, y1, s0, s1, d16, b.reshape(1, H), sg.reshape(1, H),
            bb.reshape(1, H)]
    if has_next:
        in_specs.append(pl.BlockSpec((H, H), lambda i: (0, 0)))
        out_specs += [
            pl.BlockSpec((BN_TC, HH), lambda i: (i, 0)),
            pl.BlockSpec((BN_TC, HH), lambda i: (i, 0)),
        ]
        out_shape += [
            jax.ShapeDtypeStruct((N, HH), jnp.float32),
            jax.ShapeDtypeStruct((N, HH), jnp.float32),
        ]
        args.append(Wn)

    return pl.pallas_call(
        body,
        grid=(nb,),
        in_specs=in_specs,
        out_specs=out_specs,
        out_shape=out_shape,
    )(*args)


def _tc_pool_head(h, batch3, W1, b1, W2row, b2):
    nb = N // BB

    def body(h_r, bt_r, w1_r, b1_r, w2_r, b2_r, o_r, s_acc, c_acc):
        step = pl.program_id(0)

        @pl.when(step == 0)
        def _():
            s_acc[...] = jnp.zeros_like(s_acc)
            c_acc[...] = jnp.zeros_like(c_acc)

        bt = bt_r[...].reshape(1, BB)
        gid = lax.broadcasted_iota(jnp.int32, (G, BB), 0)
        oh = (gid == bt).astype(jnp.float32)
        s_acc[...] += jnp.dot(oh, h_r[...], preferred_element_type=jnp.float32,
                     precision=lax.Precision.HIGHEST)
        c_acc[...] += jnp.broadcast_to(
            jnp.sum(oh, axis=1, keepdims=True), (G, 128))

        @pl.when(step == nb - 1)
        def _():
            s = s_acc[...]
            cnt = jnp.maximum(c_acc[...][:, :1], 1.0)
            gvec = jnp.concatenate([s / cnt, s], axis=1)
            t = jnp.dot(gvec, w1_r[...], preferred_element_type=jnp.float32,
                     precision=lax.Precision.HIGHEST)
            t = jnp.maximum(t + b1_r[...], 0.0)
            o = jnp.sum(t * w2_r[...], axis=1, keepdims=True) + b2_r[...]
            o_r[...] = o

    return pl.pallas_call(
        body,
        grid=(nb,),
        in_specs=[
            pl.BlockSpec((BB, H), lambda i: (i, 0)),
            pl.BlockSpec((1, 1, BB), lambda i: (i, 0, 0)),
            pl.BlockSpec((2 * H, H), lambda i: (0, 0)),
            pl.BlockSpec((1, H), lambda i: (0, 0)),
            pl.BlockSpec((1, H), lambda i: (0, 0)),
            pl.BlockSpec((1, 1), lambda i: (0, 0)),
        ],
        out_specs=pl.BlockSpec((G, 1), lambda i: (0, 0)),
        out_shape=jax.ShapeDtypeStruct((G, 1), jnp.float32),
        scratch_shapes=[
            pltpu.VMEM((G, H), jnp.float32),
            pltpu.VMEM((G, 128), jnp.float32),
        ],
    )(h, batch3, W1, b1.reshape(1, H), W2row, b2.reshape(1, 1))


def kernel(x, edge_index, batch, enc_W, enc_b, conv_W, conv_b, bn_g, bn_b,
           head_W1, head_b1, head_W2, head_b2):
    src = edge_index[0]
    dst = edge_index[1]
    pad = EPAD - src.shape[0]
    src_p = jnp.concatenate([src, jnp.zeros((pad,), jnp.int32)])
    dst_p = jnp.concatenate([dst, jnp.full((pad,), N, jnp.int32)])
    src2d = src_p.reshape(R, CHUNK)
    dst2d = dst_p.reshape(R, CHUNK)
    batch3 = batch.reshape(N // BB, 1, BB)
    sg = bn_g / jnp.sqrt(1.0 + BN_EPS)

    degp = _sc_degree(dst2d)
    degp0, degp1 = degp[0], degp[1]
    h, d16, y0, y1 = _tc_encode(x, degp0, degp1, enc_W, enc_b, conv_W[0])
    for i in range(L):
        s0, s1 = _sc_prop(y0, y1, src2d, dst2d)
        Wn = conv_W[i + 1] if i + 1 < L else None
        outs = _tc_combine(h, y0, y1, s0, s1, d16, conv_b[i], sg[i], bn_b[i],
                           Wn)
        if Wn is not None:
            h, y0, y1 = outs
        else:
            (h,) = outs

    o = _tc_pool_head(h, batch3, head_W1, head_b1,
                      head_W2.reshape(1, H), head_b2)
    return o[:, 0]


# speedup vs baseline: 1.0208x; 1.0070x over previous
import functools

import jax
import jax.numpy as jnp
from jax import lax
from jax.experimental import pallas as pl
from jax.experimental.pallas import tpu as pltpu
from jax.experimental.pallas import tpu_sc as plsc

N = 10000
D = 128
H = 256
HH = 128
G = 64
L = 4
BN_EPS = 1e-5

NSC = 2
NSUB = 16
CHUNK = 128

R = 2560
EPAD = R * CHUNK
RW_DEG = R // (NSC * NSUB)
RS_PROP = R // NSUB
IGRP = 8
PGRP = 16
NBUF = 2

NPAD = 10240
ZROWS = NPAD // NSUB

BN_TC = 1000
BB = 1000

_mesh = plsc.VectorSubcoreMesh(core_axis_name="c", subcore_axis_name="s")


@jax.jit
def _sc_degree(dst2d):
    @functools.partial(
        pl.kernel,
        mesh=_mesh,
        out_type=jax.ShapeDtypeStruct((NSC, NPAD, HH), jnp.float32),
        scratch_types=[
            pltpu.VMEM((IGRP, CHUNK), jnp.int32),
            pltpu.VMEM((CHUNK, HH), jnp.float32),
            pltpu.VMEM((16, HH), jnp.float32),
            pltpu.VMEM_SHARED((NPAD, HH), jnp.float32),
        ],
    )
    def deg_k(dst_hbm, o_hbm, idxb, onesb, zb, acc):
        cid = lax.axis_index("c")
        sid = lax.axis_index("s")
        ones16 = jnp.ones((16,), jnp.float32)
        zero16 = jnp.zeros((16,), jnp.float32)

        @pl.loop(0, CHUNK)
        def _(i):
            @pl.loop(0, HH, step=16)
            def _(j):
                onesb[i, pl.ds(j, 16)] = ones16

        @pl.loop(0, 16)
        def _(i):
            @pl.loop(0, HH, step=16)
            def _(j):
                zb[i, pl.ds(j, 16)] = zero16

        zbase = sid * ZROWS

        @pl.loop(0, ZROWS, step=16)
        def _(i):
            pltpu.sync_copy(zb, acc.at[pl.ds(zbase + i, 16)])

        plsc.subcore_barrier()

        wid = cid * NSUB + sid
        rowbase = wid * RW_DEG

        @pl.loop(0, RW_DEG, step=IGRP)
        def _(g):
            pltpu.sync_copy(dst_hbm.at[pl.ds(rowbase + g, IGRP)], idxb)
            for j in range(IGRP):
                pltpu.sync_copy(onesb, acc.at[idxb.at[j]], add=True)

        plsc.subcore_barrier()
        wbase = sid * ZROWS
        pltpu.sync_copy(acc.at[pl.ds(wbase, ZROWS)],
                        o_hbm.at[cid, pl.ds(wbase, ZROWS)])

    return deg_k(dst2d)


@jax.jit
def _sc_prop(y0, y1, src2d, dst2d):
    @functools.partial(
        pl.kernel,
        mesh=_mesh,
        out_type=(
            jax.ShapeDtypeStruct((NPAD, HH), jnp.float32),
            jax.ShapeDtypeStruct((NPAD, HH), jnp.float32),
        ),
        scratch_types=[
            pltpu.VMEM((PGRP, CHUNK), jnp.int32),
            pltpu.VMEM((PGRP, CHUNK), jnp.int32),
            pltpu.VMEM((PGRP, CHUNK), jnp.int32),
            pltpu.VMEM((PGRP, CHUNK), jnp.int32),
            pltpu.VMEM((CHUNK, HH), jnp.float32),
            pltpu.VMEM((CHUNK, HH), jnp.float32),
            pltpu.VMEM((16, HH), jnp.float32),
            pltpu.VMEM_SHARED((NPAD, HH), jnp.float32),
            pltpu.SemaphoreType.DMA,
            pltpu.SemaphoreType.DMA,
            pltpu.SemaphoreType.DMA,
            pltpu.SemaphoreType.DMA,
            pltpu.SemaphoreType.DMA,
            pltpu.SemaphoreType.DMA,
        ],
    )
    def prop_k(y0_hbm, y1_hbm, src_hbm, dst_hbm, s0_hbm, s1_hbm,
               sib0, dib0, sib1, dib1, r0, r1, zb, acc,
               g0, g1, s0, s1, isem, zsem):
        cid = lax.axis_index("c")
        sid = lax.axis_index("s")
        rows = [r0, r1]
        gsem = [g0, g1]
        ssem = [s0, s1]
        sibs = [sib0, sib1]
        dibs = [dib0, dib1]
        zero16 = jnp.zeros((16,), jnp.float32)

        @pl.loop(0, 16)
        def _(i):
            @pl.loop(0, HH, step=16)
            def _(j):
                zb[i, pl.ds(j, 16)] = zero16

        zbase = sid * ZROWS
        zcp = [pltpu.async_copy(zb, acc.at[pl.ds(zbase + 16 * i, 16)], zsem)
               for i in range(ZROWS // 16)]
        for c in zcp:
            c.wait()

        plsc.subcore_barrier()
        rowbase = sid * RS_PROP

        def run(y_hbm, o_hbm):
            def process2(bufs):
                M = 2 * PGRP
                gcp = [None] * M
                scp = [None] * M

                def ib(j):
                    sib, dib = bufs[0] if j < PGRP else bufs[1]
                    return sib.at[j % PGRP], dib.at[j % PGRP]

                def hook(g):
                    nxt = pl.multiple_of(
                        jnp.minimum(g + 2 * PGRP,
                                    rowbase + RS_PROP - PGRP), PGRP)
                    prefetch(nxt, 0)

                for j in range(M):
                    b = j % NBUF
                    if j >= NBUF:
                        scp[j - NBUF].wait()
                    sj, _ = ib(j)
                    gcp[j] = pltpu.async_copy(y_hbm.at[sj], rows[b], gsem[b])
                    if j >= 1:
                        pb = (j - 1) % NBUF
                        gcp[j - 1].wait()
                        _, dj = ib(j - 1)
                        scp[j - 1] = pltpu.async_copy(
                            rows[pb], acc.at[dj], ssem[pb], add=True)
                    if j == PGRP + 1:
                        hook(cur_g[0])
                lb = (M - 1) % NBUF
                gcp[M - 1].wait()
                _, dl = ib(M - 1)
                scp[M - 1] = pltpu.async_copy(rows[lb], acc.at[dl], ssem[lb],
                                              add=True)
                for j in range(M - NBUF, M):
                    scp[j].wait()

            def prefetch(row, bi):
                ca = pltpu.async_copy(src_hbm.at[pl.ds(row, PGRP)],
                                      sibs[bi], isem)
                cb = pltpu.async_copy(dst_hbm.at[pl.ds(row, PGRP)],
                                      dibs[bi], isem)
                return (ca, cb)

            def wait_idx(bi):
                pltpu.make_async_copy(src_hbm.at[pl.ds(rowbase, PGRP)],
                                      sibs[bi], isem).wait()
                pltpu.make_async_copy(dst_hbm.at[pl.ds(rowbase, PGRP)],
                                      dibs[bi], isem).wait()

            pltpu.sync_copy(src_hbm.at[pl.ds(rowbase, PGRP)], sibs[0])
            pltpu.sync_copy(dst_hbm.at[pl.ds(rowbase, PGRP)], dibs[0])
            pltpu.sync_copy(src_hbm.at[pl.ds(rowbase + PGRP, PGRP)], sibs[1])
            pltpu.sync_copy(dst_hbm.at[pl.ds(rowbase + PGRP, PGRP)], dibs[1])
            cur_g = [None]

            @pl.loop(0, RS_PROP, step=2 * PGRP)
            def _(g):
                cur_g[0] = rowbase + g
                process2(((sibs[0], dibs[0]), (sibs[1], dibs[1])))
                nxt2 = pl.multiple_of(
                    jnp.minimum(rowbase + g + 3 * PGRP,
                                rowbase + RS_PROP - PGRP), PGRP)
                prefetch(nxt2, 1)
                wait_idx(0)
                wait_idx(1)

            plsc.subcore_barrier()
            wbase = sid * ZROWS
            pltpu.sync_copy(acc.at[pl.ds(wbase, ZROWS)],
                            o_hbm.at[pl.ds(wbase, ZROWS)])

        @pl.when(cid == 0)
        def _():
            run(y0_hbm, s0_hbm)

        @pl.when(cid == 1)
        def _():
            run(y1_hbm, s1_hbm)

    return prop_k(y0, y1, src2d, dst2d)


def _tc_encode(x, degp0, degp1, enc_W, enc_b, W0):
    nb = N // BN_TC

    def body(x_r, d0_r, d1_r, ew_r, eb_r, w0_r, h_r, d16_r, y0_r, y1_r):
        xb = x_r[...]
        h = jnp.dot(xb, ew_r[...], preferred_element_type=jnp.float32,
                     precision=lax.Precision.HIGHEST)
        h = h + eb_r[...]
        deg = d0_r[...][:, :16] + d1_r[...][:, :16] + 1.0
        dv16 = lax.rsqrt(deg)
        dv = dv16[:, :1]
        h_r[...] = h
        d16_r[...] = dv16
        xw = jnp.dot(h, w0_r[...], preferred_element_type=jnp.float32,
                     precision=lax.Precision.HIGHEST)
        y = dv * xw
        y0_r[...] = y[:, :HH]
        y1_r[...] = y[:, HH:]

    return pl.pallas_call(
        body,
        grid=(nb,),
        in_specs=[
            pl.BlockSpec((BN_TC, D), lambda i: (i, 0)),
            pl.BlockSpec((BN_TC, HH), lambda i: (i, 0)),
            pl.BlockSpec((BN_TC, HH), lambda i: (i, 0)),
            pl.BlockSpec((D, H), lambda i: (0, 0)),
            pl.BlockSpec((1, H), lambda i: (0, 0)),
            pl.BlockSpec((H, H), lambda i: (0, 0)),
        ],
        out_specs=[
            pl.BlockSpec((BN_TC, H), lambda i: (i, 0)),
            pl.BlockSpec((BN_TC, 16), lambda i: (i, 0)),
            pl.BlockSpec((BN_TC, HH), lambda i: (i, 0)),
            pl.BlockSpec((BN_TC, HH), lambda i: (i, 0)),
        ],
        out_shape=[
            jax.ShapeDtypeStruct((N, H), jnp.float32),
            jax.ShapeDtypeStruct((N, 16), jnp.float32),
            jax.ShapeDtypeStruct((N, HH), jnp.float32),
            jax.ShapeDtypeStruct((N, HH), jnp.float32),
        ],
    )(x, degp0, degp1, enc_W, enc_b.reshape(1, H), W0)


def _tc_combine(h, y0, y1, s0, s1, d16, b, sg, bb, Wn):
    nb = N // BN_TC
    has_next = Wn is not None

    def body(h_r, y0_r, y1_r, s0_r, s1_r, d16_r, b_r, sg_r, bb_r, *rest):
        if has_next:
            wn_r, hn_r, yn0_r, yn1_r = rest
        else:
            (hn_r,) = rest
        dv = d16_r[...][:, :1]
        agg = jnp.concatenate([s0_r[...] + y0_r[...],
                               s1_r[...] + y1_r[...]], axis=1)
        t = dv * agg + b_r[...]
        t = jnp.maximum(t, 0.0) * sg_r[...] + bb_r[...]
        hn = h_r[...] + t
        hn_r[...] = hn
        if has_next:
            xw = jnp.dot(hn, wn_r[...], preferred_element_type=jnp.float32,
                     precision=lax.Precision.HIGHEST)
            y = dv * xw
            yn0_r[...] = y[:, :HH]
            yn1_r[...] = y[:, HH:]

    in_specs = [
        pl.BlockSpec((BN_TC, H), lambda i: (i, 0)),
        pl.BlockSpec((BN_TC, HH), lambda i: (i, 0)),
        pl.BlockSpec((BN_TC, HH), lambda i: (i, 0)),
        pl.BlockSpec((BN_TC, HH), lambda i: (i, 0)),
        pl.BlockSpec((BN_TC, HH), lambda i: (i, 0)),
        pl.BlockSpec((BN_TC, 16), lambda i: (i, 0)),
        pl.BlockSpec((1, H), lambda i: (0, 0)),
        pl.BlockSpec((1, H), lambda i: (0, 0)),
        pl.BlockSpec((1, H), lambda i: (0, 0)),
    ]
    out_specs = [pl.BlockSpec((BN_TC, H), lambda i: (i, 0))]
    out_shape = [jax.ShapeDtypeStruct((N, H), jnp.float32)]
    args = [h, y0, y1, s0, s1, d16, b.reshape(1, H), sg.reshape(1, H),
            bb.reshape(1, H)]
    if has_next:
        in_specs.append(pl.BlockSpec((H, H), lambda i: (0, 0)))
        out_specs += [
            pl.BlockSpec((BN_TC, HH), lambda i: (i, 0)),
            pl.BlockSpec((BN_TC, HH), lambda i: (i, 0)),
        ]
        out_shape += [
            jax.ShapeDtypeStruct((N, HH), jnp.float32),
            jax.ShapeDtypeStruct((N, HH), jnp.float32),
        ]
        args.append(Wn)

    return pl.pallas_call(
        body,
        grid=(nb,),
        in_specs=in_specs,
        out_specs=out_specs,
        out_shape=out_shape,
    )(*args)


def _tc_pool_head(h, batch3, W1, b1, W2row, b2):
    nb = N // BB

    def body(h_r, bt_r, w1_r, b1_r, w2_r, b2_r, o_r, s_acc, c_acc):
        step = pl.program_id(0)

        @pl.when(step == 0)
        def _():
            s_acc[...] = jnp.zeros_like(s_acc)
            c_acc[...] = jnp.zeros_like(c_acc)

        bt = bt_r[...].reshape(1, BB)
        gid = lax.broadcasted_iota(jnp.int32, (G, BB), 0)
        oh = (gid == bt).astype(jnp.float32)
        s_acc[...] += jnp.dot(oh, h_r[...], preferred_element_type=jnp.float32,
                     precision=lax.Precision.HIGHEST)
        c_acc[...] += jnp.broadcast_to(
            jnp.sum(oh, axis=1, keepdims=True), (G, 128))

        @pl.when(step == nb - 1)
        def _():
            s = s_acc[...]
            cnt = jnp.maximum(c_acc[...][:, :1], 1.0)
            gvec = jnp.concatenate([s / cnt, s], axis=1)
            t = jnp.dot(gvec, w1_r[...], preferred_element_type=jnp.float32,
                     precision=lax.Precision.HIGHEST)
            t = jnp.maximum(t + b1_r[...], 0.0)
            o = jnp.sum(t * w2_r[...], axis=1, keepdims=True) + b2_r[...]
            o_r[...] = o

    return pl.pallas_call(
        body,
        grid=(nb,),
        in_specs=[
            pl.BlockSpec((BB, H), lambda i: (i, 0)),
            pl.BlockSpec((1, 1, BB), lambda i: (i, 0, 0)),
            pl.BlockSpec((2 * H, H), lambda i: (0, 0)),
            pl.BlockSpec((1, H), lambda i: (0, 0)),
            pl.BlockSpec((1, H), lambda i: (0, 0)),
            pl.BlockSpec((1, 1), lambda i: (0, 0)),
        ],
        out_specs=pl.BlockSpec((G, 1), lambda i: (0, 0)),
        out_shape=jax.ShapeDtypeStruct((G, 1), jnp.float32),
        scratch_shapes=[
            pltpu.VMEM((G, H), jnp.float32),
            pltpu.VMEM((G, 128), jnp.float32),
        ],
    )(h, batch3, W1, b1.reshape(1, H), W2row, b2.reshape(1, 1))


def kernel(x, edge_index, batch, enc_W, enc_b, conv_W, conv_b, bn_g, bn_b,
           head_W1, head_b1, head_W2, head_b2):
    src = edge_index[0]
    dst = edge_index[1]
    pad = EPAD - src.shape[0]
    src_p = jnp.concatenate([src, jnp.zeros((pad,), jnp.int32)])
    dst_p = jnp.concatenate([dst, jnp.full((pad,), N, jnp.int32)])
    src2d = src_p.reshape(R, CHUNK)
    dst2d = dst_p.reshape(R, CHUNK)
    batch3 = batch.reshape(N // BB, 1, BB)
    sg = bn_g / jnp.sqrt(1.0 + BN_EPS)

    degp = _sc_degree(dst2d)
    degp0, degp1 = degp[0], degp[1]
    h, d16, y0, y1 = _tc_encode(x, degp0, degp1, enc_W, enc_b, conv_W[0])
    for i in range(L):
        s0, s1 = _sc_prop(y0, y1, src2d, dst2d)
        Wn = conv_W[i + 1] if i + 1 < L else None
        outs = _tc_combine(h, y0, y1, s0, s1, d16, conv_b[i], sg[i], bn_b[i],
                           Wn)
        if Wn is not None:
            h, y0, y1 = outs
        else:
            (h,) = outs

    o = _tc_pool_head(h, batch3, head_W1, head_b1,
                      head_W2.reshape(1, H), head_b2)
    return o[:, 0]


# speedup vs baseline: 1.1243x; 1.1014x over previous
import functools

import jax
import jax.numpy as jnp
from jax import lax
from jax.experimental import pallas as pl
from jax.experimental.pallas import tpu as pltpu
from jax.experimental.pallas import tpu_sc as plsc

N = 10000
D = 128
H = 256
HH = 128
G = 64
L = 4
BN_EPS = 1e-5

NSC = 2
NSUB = 16
CHUNK = 128

R = 2560
EPAD = R * CHUNK
RW_DEG = R // (NSC * NSUB)
RS_PROP = R // NSUB
IGRP = 8
PGRP = 16
NBUF = 2

NPAD = 10240
ZROWS = NPAD // NSUB

BN_TC = 1000
BB = 1000

_mesh = plsc.VectorSubcoreMesh(core_axis_name="c", subcore_axis_name="s")


@jax.jit
def _sc_degree(dst2d):
    @functools.partial(
        pl.kernel,
        mesh=_mesh,
        out_type=jax.ShapeDtypeStruct((NSC, NPAD, HH), jnp.float32),
        scratch_types=[
            pltpu.VMEM((IGRP, CHUNK), jnp.int32),
            pltpu.VMEM((CHUNK, HH), jnp.float32),
            pltpu.VMEM((16, HH), jnp.float32),
            pltpu.VMEM_SHARED((NPAD, HH), jnp.float32),
        ],
    )
    def deg_k(dst_hbm, o_hbm, idxb, onesb, zb, acc):
        cid = lax.axis_index("c")
        sid = lax.axis_index("s")
        ones16 = jnp.ones((16,), jnp.float32)
        zero16 = jnp.zeros((16,), jnp.float32)

        @pl.loop(0, CHUNK)
        def _(i):
            @pl.loop(0, HH, step=16)
            def _(j):
                onesb[i, pl.ds(j, 16)] = ones16

        @pl.loop(0, 16)
        def _(i):
            @pl.loop(0, HH, step=16)
            def _(j):
                zb[i, pl.ds(j, 16)] = zero16

        zbase = sid * ZROWS

        @pl.loop(0, ZROWS, step=16)
        def _(i):
            pltpu.sync_copy(zb, acc.at[pl.ds(zbase + i, 16)])

        plsc.subcore_barrier()

        wid = cid * NSUB + sid
        rowbase = wid * RW_DEG

        @pl.loop(0, RW_DEG, step=IGRP)
        def _(g):
            pltpu.sync_copy(dst_hbm.at[pl.ds(rowbase + g, IGRP)], idxb)
            for j in range(IGRP):
                pltpu.sync_copy(onesb, acc.at[idxb.at[j]], add=True)

        plsc.subcore_barrier()
        wbase = sid * ZROWS
        pltpu.sync_copy(acc.at[pl.ds(wbase, ZROWS)],
                        o_hbm.at[cid, pl.ds(wbase, ZROWS)])

    return deg_k(dst2d)


@jax.jit
def _sc_prop(y0, y1, src2d, dst2d):
    @functools.partial(
        pl.kernel,
        mesh=_mesh,
        out_type=(
            jax.ShapeDtypeStruct((NPAD, HH), jnp.float32),
            jax.ShapeDtypeStruct((NPAD, HH), jnp.float32),
        ),
        scratch_types=[
            pltpu.VMEM((PGRP, CHUNK), jnp.int32),
            pltpu.VMEM((PGRP, CHUNK), jnp.int32),
            pltpu.VMEM((PGRP, CHUNK), jnp.int32),
            pltpu.VMEM((PGRP, CHUNK), jnp.int32),
            pltpu.VMEM((CHUNK, HH), jnp.float32),
            pltpu.VMEM((CHUNK, HH), jnp.float32),
            pltpu.VMEM((16, HH), jnp.float32),
            pltpu.VMEM_SHARED((NPAD, HH), jnp.float32),
            pltpu.SemaphoreType.DMA,
            pltpu.SemaphoreType.DMA,
            pltpu.SemaphoreType.DMA,
            pltpu.SemaphoreType.DMA,
            pltpu.SemaphoreType.DMA,
            pltpu.SemaphoreType.DMA,
        ],
    )
    def prop_k(y0_hbm, y1_hbm, src_hbm, dst_hbm, s0_hbm, s1_hbm,
               sib0, dib0, sib1, dib1, r0, r1, zb, acc,
               g0, g1, s0, s1, isem, zsem):
        cid = lax.axis_index("c")
        sid = lax.axis_index("s")
        rows = [r0, r1]
        gsem = [g0, g1]
        ssem = [s0, s1]
        sibs = [sib0, sib1]
        dibs = [dib0, dib1]
        zero16 = jnp.zeros((16,), jnp.float32)

        @pl.loop(0, 16)
        def _(i):
            @pl.loop(0, HH, step=16)
            def _(j):
                zb[i, pl.ds(j, 16)] = zero16

        zbase = sid * ZROWS
        zcp = [pltpu.async_copy(zb, acc.at[pl.ds(zbase + 16 * i, 16)], zsem)
               for i in range(ZROWS // 16)]
        for c in zcp:
            c.wait()

        plsc.subcore_barrier()
        rowbase = sid * RS_PROP

        def run(y_hbm, o_hbm):
            def process2(bufs):
                M = 2 * PGRP
                gcp = [None] * M
                scp = [None] * M

                def ib(j):
                    sib, dib = bufs[0] if j < PGRP else bufs[1]
                    return sib.at[j % PGRP], dib.at[j % PGRP]

                def hook(g):
                    nxt = pl.multiple_of(
                        jnp.minimum(g + 2 * PGRP,
                                    rowbase + RS_PROP - PGRP), PGRP)
                    prefetch(nxt, 0)

                for j in range(M):
                    b = j % NBUF
                    if j >= NBUF:
                        scp[j - NBUF].wait()
                    sj, _ = ib(j)
                    gcp[j] = pltpu.async_copy(y_hbm.at[sj], rows[b], gsem[b])
                    if j >= 1:
                        pb = (j - 1) % NBUF
                        gcp[j - 1].wait()
                        _, dj = ib(j - 1)
                        scp[j - 1] = pltpu.async_copy(
                            rows[pb], acc.at[dj], ssem[pb], add=True)
                    if j == PGRP + 1:
                        hook(cur_g[0])
                lb = (M - 1) % NBUF
                gcp[M - 1].wait()
                _, dl = ib(M - 1)
                scp[M - 1] = pltpu.async_copy(rows[lb], acc.at[dl], ssem[lb],
                                              add=True)
                for j in range(M - NBUF, M):
                    scp[j].wait()

            def prefetch(row, bi):
                ca = pltpu.async_copy(src_hbm.at[pl.ds(row, PGRP)],
                                      sibs[bi], isem)
                cb = pltpu.async_copy(dst_hbm.at[pl.ds(row, PGRP)],
                                      dibs[bi], isem)
                return (ca, cb)

            def wait_idx(bi):
                pltpu.make_async_copy(src_hbm.at[pl.ds(rowbase, PGRP)],
                                      sibs[bi], isem).wait()
                pltpu.make_async_copy(dst_hbm.at[pl.ds(rowbase, PGRP)],
                                      dibs[bi], isem).wait()

            pltpu.sync_copy(src_hbm.at[pl.ds(rowbase, PGRP)], sibs[0])
            pltpu.sync_copy(dst_hbm.at[pl.ds(rowbase, PGRP)], dibs[0])
            pltpu.sync_copy(src_hbm.at[pl.ds(rowbase + PGRP, PGRP)], sibs[1])
            pltpu.sync_copy(dst_hbm.at[pl.ds(rowbase + PGRP, PGRP)], dibs[1])
            cur_g = [None]

            @pl.loop(0, RS_PROP, step=2 * PGRP)
            def _(g):
                cur_g[0] = rowbase + g
                process2(((sibs[0], dibs[0]), (sibs[1], dibs[1])))
                nxt2 = pl.multiple_of(
                    jnp.minimum(rowbase + g + 3 * PGRP,
                                rowbase + RS_PROP - PGRP), PGRP)
                prefetch(nxt2, 1)
                wait_idx(0)
                wait_idx(1)

            plsc.subcore_barrier()
            wbase = sid * ZROWS
            pltpu.sync_copy(acc.at[pl.ds(wbase, ZROWS)],
                            o_hbm.at[pl.ds(wbase, ZROWS)])

        @pl.when(cid == 0)
        def _():
            run(y0_hbm, s0_hbm)

        @pl.when(cid == 1)
        def _():
            run(y1_hbm, s1_hbm)

    return prop_k(y0, y1, src2d, dst2d)


def _tc_enc_h(x, enc_W, enc_b):
    nb = N // BN_TC

    def body(x_r, ew_r, eb_r, h_r):
        h = jnp.dot(x_r[...], ew_r[...], preferred_element_type=jnp.float32,
                    precision=lax.Precision.HIGHEST)
        h_r[...] = h + eb_r[...]

    return pl.pallas_call(
        body,
        grid=(nb,),
        in_specs=[
            pl.BlockSpec((BN_TC, D), lambda i: (i, 0)),
            pl.BlockSpec((D, H), lambda i: (0, 0)),
            pl.BlockSpec((1, H), lambda i: (0, 0)),
        ],
        out_specs=pl.BlockSpec((BN_TC, H), lambda i: (i, 0)),
        out_shape=jax.ShapeDtypeStruct((N, H), jnp.float32),
    )(x, enc_W, enc_b.reshape(1, H))


def _tc_prep(h, degp0, degp1, W0):
    nb = N // BN_TC

    def body(h_r, d0_r, d1_r, w0_r, d16_r, y0_r, y1_r):
        deg = d0_r[...][:, :16] + d1_r[...][:, :16] + 1.0
        dv16 = lax.rsqrt(deg)
        dv = dv16[:, :1]
        d16_r[...] = dv16
        xw = jnp.dot(h_r[...], w0_r[...], preferred_element_type=jnp.float32,
                     precision=lax.Precision.HIGHEST)
        y = dv * xw
        y0_r[...] = y[:, :HH]
        y1_r[...] = y[:, HH:]

    return pl.pallas_call(
        body,
        grid=(nb,),
        in_specs=[
            pl.BlockSpec((BN_TC, H), lambda i: (i, 0)),
            pl.BlockSpec((BN_TC, HH), lambda i: (i, 0)),
            pl.BlockSpec((BN_TC, HH), lambda i: (i, 0)),
            pl.BlockSpec((H, H), lambda i: (0, 0)),
        ],
        out_specs=[
            pl.BlockSpec((BN_TC, 16), lambda i: (i, 0)),
            pl.BlockSpec((BN_TC, HH), lambda i: (i, 0)),
            pl.BlockSpec((BN_TC, HH), lambda i: (i, 0)),
        ],
        out_shape=[
            jax.ShapeDtypeStruct((N, 16), jnp.float32),
            jax.ShapeDtypeStruct((N, HH), jnp.float32),
            jax.ShapeDtypeStruct((N, HH), jnp.float32),
        ],
    )(h, degp0, degp1, W0)


def _tc_combine(h, y0, y1, s0, s1, d16, b, sg, bb, Wn):
    nb = N // BN_TC
    has_next = Wn is not None

    def body(h_r, y0_r, y1_r, s0_r, s1_r, d16_r, b_r, sg_r, bb_r, *rest):
        if has_next:
            wn_r, hn_r, yn0_r, yn1_r = rest
        else:
            (hn_r,) = rest
        dv = d16_r[...][:, :1]
        agg = jnp.concatenate([s0_r[...] + y0_r[...],
                               s1_r[...] + y1_r[...]], axis=1)
        t = dv * agg + b_r[...]
        t = jnp.maximum(t, 0.0) * sg_r[...] + bb_r[...]
        hn = h_r[...] + t
        hn_r[...] = hn
        if has_next:
            xw = jnp.dot(hn, wn_r[...], preferred_element_type=jnp.float32,
                     precision=lax.Precision.HIGHEST)
            y = dv * xw
            yn0_r[...] = y[:, :HH]
            yn1_r[...] = y[:, HH:]

    in_specs = [
        pl.BlockSpec((BN_TC, H), lambda i: (i, 0)),
        pl.BlockSpec((BN_TC, HH), lambda i: (i, 0)),
        pl.BlockSpec((BN_TC, HH), lambda i: (i, 0)),
        pl.BlockSpec((BN_TC, HH), lambda i: (i, 0)),
        pl.BlockSpec((BN_TC, HH), lambda i: (i, 0)),
        pl.BlockSpec((BN_TC, 16), lambda i: (i, 0)),
        pl.BlockSpec((1, H), lambda i: (0, 0)),
        pl.BlockSpec((1, H), lambda i: (0, 0)),
        pl.BlockSpec((1, H), lambda i: (0, 0)),
    ]
    out_specs = [pl.BlockSpec((BN_TC, H), lambda i: (i, 0))]
    out_shape = [jax.ShapeDtypeStruct((N, H), jnp.float32)]
    args = [h, y0, y1, s0, s1, d16, b.reshape(1, H), sg.reshape(1, H),
            bb.reshape(1, H)]
    if has_next:
        in_specs.append(pl.BlockSpec((H, H), lambda i: (0, 0)))
        out_specs += [
            pl.BlockSpec((BN_TC, HH), lambda i: (i, 0)),
            pl.BlockSpec((BN_TC, HH), lambda i: (i, 0)),
        ]
        out_shape += [
            jax.ShapeDtypeStruct((N, HH), jnp.float32),
            jax.ShapeDtypeStruct((N, HH), jnp.float32),
        ]
        args.append(Wn)

    return pl.pallas_call(
        body,
        grid=(nb,),
        in_specs=in_specs,
        out_specs=out_specs,
        out_shape=out_shape,
    )(*args)


def _tc_pool_head(h, batch3, W1, b1, W2row, b2):
    nb = N // BB

    def body(h_r, bt_r, w1_r, b1_r, w2_r, b2_r, o_r, s_acc, c_acc):
        step = pl.program_id(0)

        @pl.when(step == 0)
        def _():
            s_acc[...] = jnp.zeros_like(s_acc)
            c_acc[...] = jnp.zeros_like(c_acc)

        bt = bt_r[...].reshape(1, BB)
        gid = lax.broadcasted_iota(jnp.int32, (G, BB), 0)
        oh = (gid == bt).astype(jnp.float32)
        s_acc[...] += jnp.dot(oh, h_r[...], preferred_element_type=jnp.float32,
                     precision=lax.Precision.HIGHEST)
        c_acc[...] += jnp.broadcast_to(
            jnp.sum(oh, axis=1, keepdims=True), (G, 128))

        @pl.when(step == nb - 1)
        def _():
            s = s_acc[...]
            cnt = jnp.maximum(c_acc[...][:, :1], 1.0)
            gvec = jnp.concatenate([s / cnt, s], axis=1)
            t = jnp.dot(gvec, w1_r[...], preferred_element_type=jnp.float32,
                     precision=lax.Precision.HIGHEST)
            t = jnp.maximum(t + b1_r[...], 0.0)
            o = jnp.sum(t * w2_r[...], axis=1, keepdims=True) + b2_r[...]
            o_r[...] = o

    return pl.pallas_call(
        body,
        grid=(nb,),
        in_specs=[
            pl.BlockSpec((BB, H), lambda i: (i, 0)),
            pl.BlockSpec((1, 1, BB), lambda i: (i, 0, 0)),
            pl.BlockSpec((2 * H, H), lambda i: (0, 0)),
            pl.BlockSpec((1, H), lambda i: (0, 0)),
            pl.BlockSpec((1, H), lambda i: (0, 0)),
            pl.BlockSpec((1, 1), lambda i: (0, 0)),
        ],
        out_specs=pl.BlockSpec((G, 1), lambda i: (0, 0)),
        out_shape=jax.ShapeDtypeStruct((G, 1), jnp.float32),
        scratch_shapes=[
            pltpu.VMEM((G, H), jnp.float32),
            pltpu.VMEM((G, 128), jnp.float32),
        ],
    )(h, batch3, W1, b1.reshape(1, H), W2row, b2.reshape(1, 1))


def kernel(x, edge_index, batch, enc_W, enc_b, conv_W, conv_b, bn_g, bn_b,
           head_W1, head_b1, head_W2, head_b2):
    src = edge_index[0]
    dst = edge_index[1]
    pad = EPAD - src.shape[0]
    src_p = jnp.concatenate([src, jnp.zeros((pad,), jnp.int32)])
    dst_p = jnp.concatenate([dst, jnp.full((pad,), N, jnp.int32)])
    src2d = src_p.reshape(R, CHUNK)
    dst2d = dst_p.reshape(R, CHUNK)
    batch3 = batch.reshape(N // BB, 1, BB)
    sg = bn_g / jnp.sqrt(1.0 + BN_EPS)

    degp = _sc_degree(dst2d)
    h = _tc_enc_h(x, enc_W, enc_b)
    degp0, degp1 = degp[0], degp[1]
    d16, y0, y1 = _tc_prep(h, degp0, degp1, conv_W[0])
    for i in range(L):
        s0, s1 = _sc_prop(y0, y1, src2d, dst2d)
        Wn = conv_W[i + 1] if i + 1 < L else None
        outs = _tc_combine(h, y0, y1, s0, s1, d16, conv_b[i], sg[i], bn_b[i],
                           Wn)
        if Wn is not None:
            h, y0, y1 = outs
        else:
            (h,) = outs

    o = _tc_pool_head(h, batch3, head_W1, head_b1,
                      head_W2.reshape(1, H), head_b2)
    return o[:, 0]


# speedup vs baseline: 1.1301x; 1.0052x over previous
import functools

import jax
import jax.numpy as jnp
from jax import lax
from jax.experimental import pallas as pl
from jax.experimental.pallas import tpu as pltpu
from jax.experimental.pallas import tpu_sc as plsc

N = 10000
D = 128
H = 256
HH = 128
G = 64
L = 4
BN_EPS = 1e-5

NSC = 2
NSUB = 16
CHUNK = 128

R = 2560
EPAD = R * CHUNK
RW_DEG = R // (NSC * NSUB)
RS_PROP = R // NSUB
IGRP = 8
PGRP = 16
NBUF = 2

NPAD = 10240
ZROWS = NPAD // NSUB

BN_TC = 1000
BB = 1000

_mesh = plsc.VectorSubcoreMesh(core_axis_name="c", subcore_axis_name="s")


@jax.jit
def _sc_degree(dst2d):
    @functools.partial(
        pl.kernel,
        mesh=_mesh,
        out_type=jax.ShapeDtypeStruct((NSC, NPAD, HH), jnp.float32),
        scratch_types=[
            pltpu.VMEM((IGRP, CHUNK), jnp.int32),
            pltpu.VMEM((CHUNK, HH), jnp.float32),
            pltpu.VMEM((16, HH), jnp.float32),
            pltpu.VMEM_SHARED((NPAD, HH), jnp.float32),
        ],
    )
    def deg_k(dst_hbm, o_hbm, idxb, onesb, zb, acc):
        cid = lax.axis_index("c")
        sid = lax.axis_index("s")
        ones16 = jnp.ones((16,), jnp.float32)
        zero16 = jnp.zeros((16,), jnp.float32)

        @pl.loop(0, CHUNK)
        def _(i):
            @pl.loop(0, HH, step=16)
            def _(j):
                onesb[i, pl.ds(j, 16)] = ones16

        @pl.loop(0, 16)
        def _(i):
            @pl.loop(0, HH, step=16)
            def _(j):
                zb[i, pl.ds(j, 16)] = zero16

        zbase = sid * ZROWS

        @pl.loop(0, ZROWS, step=16)
        def _(i):
            pltpu.sync_copy(zb, acc.at[pl.ds(zbase + i, 16)])

        plsc.subcore_barrier()

        wid = cid * NSUB + sid
        rowbase = wid * RW_DEG

        @pl.loop(0, RW_DEG, step=IGRP)
        def _(g):
            pltpu.sync_copy(dst_hbm.at[pl.ds(rowbase + g, IGRP)], idxb)
            for j in range(IGRP):
                pltpu.sync_copy(onesb, acc.at[idxb.at[j]], add=True)

        plsc.subcore_barrier()
        wbase = sid * ZROWS
        pltpu.sync_copy(acc.at[pl.ds(wbase, ZROWS)],
                        o_hbm.at[cid, pl.ds(wbase, ZROWS)])

    return deg_k(dst2d)


@jax.jit
def _sc_prop(y0, y1, src2d, dst2d):
    @functools.partial(
        pl.kernel,
        mesh=_mesh,
        out_type=(
            jax.ShapeDtypeStruct((NPAD, HH), jnp.float32),
            jax.ShapeDtypeStruct((NPAD, HH), jnp.float32),
        ),
        scratch_types=[
            pltpu.VMEM((PGRP, CHUNK), jnp.int32),
            pltpu.VMEM((PGRP, CHUNK), jnp.int32),
            pltpu.VMEM((PGRP, CHUNK), jnp.int32),
            pltpu.VMEM((PGRP, CHUNK), jnp.int32),
            pltpu.VMEM((CHUNK, HH), jnp.float32),
            pltpu.VMEM((CHUNK, HH), jnp.float32),
            pltpu.VMEM((16, HH), jnp.float32),
            pltpu.VMEM_SHARED((NPAD, HH), jnp.float32),
            pltpu.SemaphoreType.DMA,
            pltpu.SemaphoreType.DMA,
            pltpu.SemaphoreType.DMA,
            pltpu.SemaphoreType.DMA,
            pltpu.SemaphoreType.DMA,
            pltpu.SemaphoreType.DMA,
        ],
    )
    def prop_k(y0_hbm, y1_hbm, src_hbm, dst_hbm, s0_hbm, s1_hbm,
               sib0, dib0, sib1, dib1, r0, r1, zb, acc,
               g0, g1, s0, s1, isem, zsem):
        cid = lax.axis_index("c")
        sid = lax.axis_index("s")
        rows = [r0, r1]
        gsem = [g0, g1]
        ssem = [s0, s1]
        sibs = [sib0, sib1]
        dibs = [dib0, dib1]
        zero16 = jnp.zeros((16,), jnp.float32)

        @pl.loop(0, 16)
        def _(i):
            @pl.loop(0, HH, step=16)
            def _(j):
                zb[i, pl.ds(j, 16)] = zero16

        zbase = sid * ZROWS
        zcp = [pltpu.async_copy(zb, acc.at[pl.ds(zbase + 16 * i, 16)], zsem)
               for i in range(ZROWS // 16)]
        for c in zcp:
            c.wait()

        plsc.subcore_barrier()
        rowbase = sid * RS_PROP

        def run(y_hbm, o_hbm):
            def process2(bufs):
                M = 2 * PGRP
                gcp = [None] * M
                scp = [None] * M

                def ib(j):
                    sib, dib = bufs[0] if j < PGRP else bufs[1]
                    return sib.at[j % PGRP], dib.at[j % PGRP]

                def hook(g):
                    nxt = pl.multiple_of(
                        jnp.minimum(g + 2 * PGRP,
                                    rowbase + RS_PROP - PGRP), PGRP)
                    prefetch(nxt, 0)

                for j in range(M):
                    b = j % NBUF
                    if j >= NBUF:
                        scp[j - NBUF].wait()
                    sj, _ = ib(j)
                    gcp[j] = pltpu.async_copy(y_hbm.at[sj], rows[b], gsem[b])
                    if j >= 1:
                        pb = (j - 1) % NBUF
                        gcp[j - 1].wait()
                        _, dj = ib(j - 1)
                        scp[j - 1] = pltpu.async_copy(
                            rows[pb], acc.at[dj], ssem[pb], add=True)
                    if j == PGRP + 1:
                        hook(cur_g[0])
                lb = (M - 1) % NBUF
                gcp[M - 1].wait()
                _, dl = ib(M - 1)
                scp[M - 1] = pltpu.async_copy(rows[lb], acc.at[dl], ssem[lb],
                                              add=True)
                for j in range(M - NBUF, M):
                    scp[j].wait()

            def prefetch(row, bi):
                ca = pltpu.async_copy(src_hbm.at[pl.ds(row, PGRP)],
                                      sibs[bi], isem)
                cb = pltpu.async_copy(dst_hbm.at[pl.ds(row, PGRP)],
                                      dibs[bi], isem)
                return (ca, cb)

            def wait_idx(bi):
                pltpu.make_async_copy(src_hbm.at[pl.ds(rowbase, PGRP)],
                                      sibs[bi], isem).wait()
                pltpu.make_async_copy(dst_hbm.at[pl.ds(rowbase, PGRP)],
                                      dibs[bi], isem).wait()

            pltpu.sync_copy(src_hbm.at[pl.ds(rowbase, PGRP)], sibs[0])
            pltpu.sync_copy(dst_hbm.at[pl.ds(rowbase, PGRP)], dibs[0])
            pltpu.sync_copy(src_hbm.at[pl.ds(rowbase + PGRP, PGRP)], sibs[1])
            pltpu.sync_copy(dst_hbm.at[pl.ds(rowbase + PGRP, PGRP)], dibs[1])
            cur_g = [None]

            @pl.loop(0, RS_PROP, step=2 * PGRP)
            def _(g):
                cur_g[0] = rowbase + g
                process2(((sibs[0], dibs[0]), (sibs[1], dibs[1])))
                nxt2 = pl.multiple_of(
                    jnp.minimum(rowbase + g + 3 * PGRP,
                                rowbase + RS_PROP - PGRP), PGRP)
                prefetch(nxt2, 1)
                wait_idx(0)
                wait_idx(1)

            plsc.subcore_barrier()
            wbase = sid * ZROWS
            pltpu.sync_copy(acc.at[pl.ds(wbase, ZROWS)],
                            o_hbm.at[pl.ds(wbase, ZROWS)])

        @pl.when(cid == 0)
        def _():
            run(y0_hbm, s0_hbm)

        @pl.when(cid == 1)
        def _():
            run(y1_hbm, s1_hbm)

    return prop_k(y0, y1, src2d, dst2d)


def _tc_enc_h(x, enc_W, enc_b):
    nb = N // BN_TC

    def body(x_r, ew_r, eb_r, h_r):
        h = jnp.dot(x_r[...], ew_r[...], preferred_element_type=jnp.float32)
        h_r[...] = h + eb_r[...]

    return pl.pallas_call(
        body,
        grid=(nb,),
        in_specs=[
            pl.BlockSpec((BN_TC, D), lambda i: (i, 0)),
            pl.BlockSpec((D, H), lambda i: (0, 0)),
            pl.BlockSpec((1, H), lambda i: (0, 0)),
        ],
        out_specs=pl.BlockSpec((BN_TC, H), lambda i: (i, 0)),
        out_shape=jax.ShapeDtypeStruct((N, H), jnp.float32),
    )(x, enc_W, enc_b.reshape(1, H))


def _tc_prep(h, degp0, degp1, W0):
    nb = N // BN_TC

    def body(h_r, d0_r, d1_r, w0_r, d16_r, y0_r, y1_r):
        deg = d0_r[...][:, :16] + d1_r[...][:, :16] + 1.0
        dv16 = lax.rsqrt(deg)
        dv = dv16[:, :1]
        d16_r[...] = dv16
        xw = jnp.dot(h_r[...], w0_r[...], preferred_element_type=jnp.float32)
        y = dv * xw
        y0_r[...] = y[:, :HH]
        y1_r[...] = y[:, HH:]

    return pl.pallas_call(
        body,
        grid=(nb,),
        in_specs=[
            pl.BlockSpec((BN_TC, H), lambda i: (i, 0)),
            pl.BlockSpec((BN_TC, HH), lambda i: (i, 0)),
            pl.BlockSpec((BN_TC, HH), lambda i: (i, 0)),
            pl.BlockSpec((H, H), lambda i: (0, 0)),
        ],
        out_specs=[
            pl.BlockSpec((BN_TC, 16), lambda i: (i, 0)),
            pl.BlockSpec((BN_TC, HH), lambda i: (i, 0)),
            pl.BlockSpec((BN_TC, HH), lambda i: (i, 0)),
        ],
        out_shape=[
            jax.ShapeDtypeStruct((N, 16), jnp.float32),
            jax.ShapeDtypeStruct((N, HH), jnp.float32),
            jax.ShapeDtypeStruct((N, HH), jnp.float32),
        ],
    )(h, degp0, degp1, W0)


def _tc_combine(h, y0, y1, s0, s1, d16, b, sg, bb, Wn):
    nb = N // BN_TC
    has_next = Wn is not None

    def body(h_r, y0_r, y1_r, s0_r, s1_r, d16_r, b_r, sg_r, bb_r, *rest):
        if has_next:
            wn_r, hn_r, yn0_r, yn1_r = rest
        else:
            (hn_r,) = rest
        dv = d16_r[...][:, :1]
        agg = jnp.concatenate([s0_r[...] + y0_r[...],
                               s1_r[...] + y1_r[...]], axis=1)
        t = dv * agg + b_r[...]
        t = jnp.maximum(t, 0.0) * sg_r[...] + bb_r[...]
        hn = h_r[...] + t
        hn_r[...] = hn
        if has_next:
            xw = jnp.dot(hn, wn_r[...], preferred_element_type=jnp.float32)
            y = dv * xw
            yn0_r[...] = y[:, :HH]
            yn1_r[...] = y[:, HH:]

    in_specs = [
        pl.BlockSpec((BN_TC, H), lambda i: (i, 0)),
        pl.BlockSpec((BN_TC, HH), lambda i: (i, 0)),
        pl.BlockSpec((BN_TC, HH), lambda i: (i, 0)),
        pl.BlockSpec((BN_TC, HH), lambda i: (i, 0)),
        pl.BlockSpec((BN_TC, HH), lambda i: (i, 0)),
        pl.BlockSpec((BN_TC, 16), lambda i: (i, 0)),
        pl.BlockSpec((1, H), lambda i: (0, 0)),
        pl.BlockSpec((1, H), lambda i: (0, 0)),
        pl.BlockSpec((1, H), lambda i: (0, 0)),
    ]
    out_specs = [pl.BlockSpec((BN_TC, H), lambda i: (i, 0))]
    out_shape = [jax.ShapeDtypeStruct((N, H), jnp.float32)]
    args = [h, y0, y1, s0, s1, d16, b.reshape(1, H), sg.reshape(1, H),
            bb.reshape(1, H)]
    if has_next:
        in_specs.append(pl.BlockSpec((H, H), lambda i: (0, 0)))
        out_specs += [
            pl.BlockSpec((BN_TC, HH), lambda i: (i, 0)),
            pl.BlockSpec((BN_TC, HH), lambda i: (i, 0)),
        ]
        out_shape += [
            jax.ShapeDtypeStruct((N, HH), jnp.float32),
            jax.ShapeDtypeStruct((N, HH), jnp.float32),
        ]
        args.append(Wn)

    return pl.pallas_call(
        body,
        grid=(nb,),
        in_specs=in_specs,
        out_specs=out_specs,
        out_shape=out_shape,
    )(*args)


def _tc_pool_head(h, batch3, W1, b1, W2row, b2):
    nb = N // BB

    def body(h_r, bt_r, w1_r, b1_r, w2_r, b2_r, o_r, s_acc, c_acc):
        step = pl.program_id(0)

        @pl.when(step == 0)
        def _():
            s_acc[...] = jnp.zeros_like(s_acc)
            c_acc[...] = jnp.zeros_like(c_acc)

        bt = bt_r[...].reshape(1, BB)
        gid = lax.broadcasted_iota(jnp.int32, (G, BB), 0)
        oh = (gid == bt).astype(jnp.float32)
        s_acc[...] += jnp.dot(oh, h_r[...], preferred_element_type=jnp.float32)
        c_acc[...] += jnp.broadcast_to(
            jnp.sum(oh, axis=1, keepdims=True), (G, 128))

        @pl.when(step == nb - 1)
        def _():
            s = s_acc[...]
            cnt = jnp.maximum(c_acc[...][:, :1], 1.0)
            gvec = jnp.concatenate([s / cnt, s], axis=1)
            t = jnp.dot(gvec, w1_r[...], preferred_element_type=jnp.float32)
            t = jnp.maximum(t + b1_r[...], 0.0)
            o = jnp.sum(t * w2_r[...], axis=1, keepdims=True) + b2_r[...]
            o_r[...] = o

    return pl.pallas_call(
        body,
        grid=(nb,),
        in_specs=[
            pl.BlockSpec((BB, H), lambda i: (i, 0)),
            pl.BlockSpec((1, 1, BB), lambda i: (i, 0, 0)),
            pl.BlockSpec((2 * H, H), lambda i: (0, 0)),
            pl.BlockSpec((1, H), lambda i: (0, 0)),
            pl.BlockSpec((1, H), lambda i: (0, 0)),
            pl.BlockSpec((1, 1), lambda i: (0, 0)),
        ],
        out_specs=pl.BlockSpec((G, 1), lambda i: (0, 0)),
        out_shape=jax.ShapeDtypeStruct((G, 1), jnp.float32),
        scratch_shapes=[
            pltpu.VMEM((G, H), jnp.float32),
            pltpu.VMEM((G, 128), jnp.float32),
        ],
    )(h, batch3, W1, b1.reshape(1, H), W2row, b2.reshape(1, 1))


def kernel(x, edge_index, batch, enc_W, enc_b, conv_W, conv_b, bn_g, bn_b,
           head_W1, head_b1, head_W2, head_b2):
    src = edge_index[0]
    dst = edge_index[1]
    pad = EPAD - src.shape[0]
    src_p = jnp.concatenate([src, jnp.zeros((pad,), jnp.int32)])
    dst_p = jnp.concatenate([dst, jnp.full((pad,), N, jnp.int32)])
    src2d = src_p.reshape(R, CHUNK)
    dst2d = dst_p.reshape(R, CHUNK)
    batch3 = batch.reshape(N // BB, 1, BB)
    sg = bn_g / jnp.sqrt(1.0 + BN_EPS)

    degp = _sc_degree(dst2d)
    h = _tc_enc_h(x, enc_W, enc_b)
    degp0, degp1 = degp[0], degp[1]
    d16, y0, y1 = _tc_prep(h, degp0, degp1, conv_W[0])
    for i in range(L):
        s0, s1 = _sc_prop(y0, y1, src2d, dst2d)
        Wn = conv_W[i + 1] if i + 1 < L else None
        outs = _tc_combine(h, y0, y1, s0, s1, d16, conv_b[i], sg[i], bn_b[i],
                           Wn)
        if Wn is not None:
            h, y0, y1 = outs
        else:
            (h,) = outs

    o = _tc_pool_head(h, batch3, head_W1, head_b1,
                      head_W2.reshape(1, H), head_b2)
    return o[:, 0]
